# baseline (speedup 1.0000x reference)
import functools

import jax
import jax.numpy as jnp
from jax import lax
from jax.experimental import pallas as pl
from jax.experimental.pallas import tpu as pltpu
from jax.experimental.pallas import tpu_sc as plsc

INPUT_DIM = 1000
OUTPUT_DIM = 64
BATCH = 4096

NUM_CORES = 2
NUM_SUBCORES = 16
NUM_WORKERS = NUM_CORES * NUM_SUBCORES
CHUNK = 8

NPART = 4
PART = BATCH // NPART

RBLK = 128
WIDE = 2 * OUTPUT_DIM


def _build_sc_gather(batch_part):
  b_per_w = batch_part // NUM_WORKERS
  nchunk = b_per_w // CHUNK
  mesh = plsc.VectorSubcoreMesh(core_axis_name="c", subcore_axis_name="s")

  @functools.partial(
      pl.kernel,
      mesh=mesh,
      out_type=jax.ShapeDtypeStruct((batch_part, OUTPUT_DIM // 2, WIDE),
                                    jnp.float32),
      scratch_types=[
          pltpu.VMEM((b_per_w,), jnp.int32),
          pltpu.VMEM((CHUNK, OUTPUT_DIM // 2, WIDE), jnp.float32),
          pltpu.VMEM((CHUNK, OUTPUT_DIM // 2, WIDE), jnp.float32),
          pltpu.SemaphoreType.DMA,
          pltpu.SemaphoreType.DMA,
          pltpu.SemaphoreType.DMA,
          pltpu.SemaphoreType.DMA,
      ],
  )
  def gather_kernel(idx_hbm, table_hbm, out_hbm, idx_v, buf0, buf1,
                    gsem0, gsem1, ssem0, ssem1):
    wid = lax.axis_index("s") * NUM_CORES + lax.axis_index("c")
    base = wid * b_per_w
    pltpu.sync_copy(idx_hbm.at[pl.ds(base, b_per_w)], idx_v)

    bufs = (buf0, buf1)
    gsems = (gsem0, gsem1)
    ssems = (ssem0, ssem1)

    def gather(g, b):
      return pltpu.async_copy(
          table_hbm.at[idx_v.at[pl.ds(g * CHUNK, CHUNK)]], bufs[b], gsems[b])

    def scatter(g, b):
      return pltpu.async_copy(
          bufs[b], out_hbm.at[pl.ds(base + g * CHUNK, CHUNK)], ssems[b])

    gd = [None] * nchunk
    sd = [None] * nchunk
    gd[0] = gather(0, 0)
    for g in range(nchunk):
      b = g % 2
      gd[g].wait()
      sd[g] = scatter(g, b)
      if g + 1 < nchunk:
        if g >= 1:
          sd[g - 1].wait()
        gd[g + 1] = gather(g + 1, 1 - b)
    if nchunk >= 2:
      sd[nchunk - 2].wait()
    sd[nchunk - 1].wait()

  return gather_kernel


_sc_gather_part = _build_sc_gather(PART)

_OUT_SHAPE = jax.ShapeDtypeStruct((BATCH, OUTPUT_DIM, OUTPUT_DIM),
                                  jnp.float32)


def _interleave(wide):
  left = wide[:, :, :OUTPUT_DIM]
  right = wide[:, :, OUTPUT_DIM:]
  return jnp.stack([left, right], axis=2).reshape(
      wide.shape[0], OUTPUT_DIM, OUTPUT_DIM)


def _build_tc_fill(part_k):
  grid = (PART // RBLK,)
  blk0 = part_k * PART // RBLK
  wide_spec = pl.BlockSpec((RBLK, OUTPUT_DIM // 2, WIDE),
                           lambda i: (i, 0, 0))
  out_spec = pl.BlockSpec((RBLK, OUTPUT_DIM, OUTPUT_DIM),
                          lambda i: (blk0 + i, 0, 0))
  if part_k == 0:
    def body0(wide_ref, out_ref):
      out_ref[...] = _interleave(wide_ref[...])
    return pl.pallas_call(
        body0, grid=grid, in_specs=[wide_spec], out_specs=out_spec,
        out_shape=_OUT_SHAPE)

  def body(prev_ref, wide_ref, out_ref):
    del prev_ref
    out_ref[...] = _interleave(wide_ref[...])
  return pl.pallas_call(
      body, grid=grid,
      in_specs=[pl.BlockSpec(memory_space=pl.ANY), wide_spec],
      out_specs=out_spec,
      out_shape=_OUT_SHAPE,
      input_output_aliases={0: 0})


_tc_fill = [_build_tc_fill(k) for k in range(NPART)]


def kernel(inputs, embeddings):
  table = embeddings.reshape(INPUT_DIM, OUTPUT_DIM // 2, WIDE)
  out = None
  for k in range(NPART):
    wide = _sc_gather_part(
        lax.slice(inputs, (k * PART,), ((k + 1) * PART,)), table)
    out = _tc_fill[k](wide) if k == 0 else _tc_fill[k](out, wide)
  return out


# speedup vs baseline: 2.1341x; 2.1341x over previous
import functools

import jax
import jax.numpy as jnp
from jax import lax
from jax.experimental import pallas as pl
from jax.experimental.pallas import tpu as pltpu
from jax.experimental.pallas import tpu_sc as plsc

INPUT_DIM = 1000
OUTPUT_DIM = 64
ROW = OUTPUT_DIM * OUTPUT_DIM
BATCH = 4096

NUM_CORES = 2
NUM_SUBCORES = 16
NUM_WORKERS = NUM_CORES * NUM_SUBCORES
B_PER_W = BATCH // NUM_WORKERS
CHUNK = 8
NCHUNK = B_PER_W // CHUNK
NBUF = 3


def _build():
  mesh = plsc.VectorSubcoreMesh(core_axis_name="c", subcore_axis_name="s")

  @functools.partial(
      pl.kernel,
      mesh=mesh,
      out_type=jax.ShapeDtypeStruct((BATCH, ROW), jnp.float32),
      scratch_types=(
          [pltpu.VMEM((B_PER_W,), jnp.int32)]
          + [pltpu.VMEM((CHUNK, ROW), jnp.float32)] * NBUF
          + [pltpu.SemaphoreType.DMA] * (2 * NBUF)
      ),
  )
  def gather_kernel(idx_hbm, table_hbm, out_hbm, idx_v, *rest):
    bufs = rest[:NBUF]
    gsems = rest[NBUF:2 * NBUF]
    ssems = rest[2 * NBUF:]
    wid = lax.axis_index("s") * NUM_CORES + lax.axis_index("c")
    base = wid * B_PER_W
    pltpu.sync_copy(idx_hbm.at[pl.ds(base, B_PER_W)], idx_v)

    def gather(g, b):
      return pltpu.async_copy(
          table_hbm.at[idx_v.at[pl.ds(g * CHUNK, CHUNK)]], bufs[b], gsems[b])

    def scatter(g, b):
      return pltpu.async_copy(
          bufs[b], out_hbm.at[pl.ds(base + g * CHUNK, CHUNK)], ssems[b])

    gd = [None] * NCHUNK
    sd = [None] * NCHUNK
    gd[0] = gather(0, 0)
    gd[1] = gather(1, 1)
    for g in range(NCHUNK):
      b = g % NBUF
      gd[g].wait()
      sd[g] = scatter(g, b)
      if g + 2 < NCHUNK:
        if g >= 1:
          sd[g - 1].wait()
        gd[g + 2] = gather(g + 2, (g + 2) % NBUF)
    sd[NCHUNK - 3].wait()
    sd[NCHUNK - 2].wait()
    sd[NCHUNK - 1].wait()

  return gather_kernel


_gather = _build()


def kernel(inputs, embeddings):
  table = embeddings.reshape(INPUT_DIM, ROW)
  out = _gather(inputs, table)
  return out.reshape(BATCH, OUTPUT_DIM, OUTPUT_DIM)
